# baseline (speedup 1.0000x reference)
import functools

import jax
import jax.numpy as jnp
from jax import lax
from jax.experimental import pallas as pl
from jax.experimental.pallas import tpu as pltpu
from jax.experimental.pallas import tpu_sc as plsc

N = 10000
E = 320000
D = 128
NP = 10240
NC = 2
NS = 16
NW = NC * NS
B = 128
EROWS = 2560
EP = EROWS * B
RPT = NP // NS
RING = 2
CHUNK = 16
NBAT0 = 160
NBAT1 = 0
NROWS0 = NS * NBAT0
NBAT_S = 160
RING2 = 8
NOUT2 = NBAT_S // RING2

_mesh = plsc.VectorSubcoreMesh(core_axis_name="c", subcore_axis_name="s")


def _zero_fill_2d(ref, rows):
    def row(i, _):
        def col(j, _):
            ref[i, pl.ds(j * 16, 16)] = jnp.zeros((16,), jnp.float32)
            return 0
        return lax.fori_loop(0, D // 16, col, 0)
    lax.fori_loop(0, rows, row, 0)


def _zero_fill_1d(ref, n):
    def body(j, _):
        ref[pl.ds(j * 16, 16)] = jnp.zeros((16,), jnp.float32)
        return 0
    lax.fori_loop(0, n // 16, body, 0)


@functools.partial(
    pl.kernel,
    out_type=(
        jax.ShapeDtypeStruct((NC, NP, D), jnp.float32),
        jax.ShapeDtypeStruct((NC, NP), jnp.float32),
    ),
    mesh=_mesh,
    scratch_types=(
        pltpu.VMEM((CHUNK, B), jnp.int32),
        pltpu.VMEM((CHUNK, B), jnp.int32),
        tuple(pltpu.VMEM((B, D), jnp.float32) for _ in range(RING)),
        pltpu.VMEM((B,), jnp.float32),
        pltpu.VMEM((RPT,), jnp.float32),
        pltpu.VMEM_SHARED((NP, D), jnp.float32),
        pltpu.VMEM_SHARED((NP,), jnp.float32),
        pltpu.SemaphoreType.DMA,
    ),
)
def _sc_agg_wide(src_hbm, dst_hbm, x_hbm, agg_out, cnt_out,
                 src_v, dst_v, rows_bufs, ones_v, zrow_v,
                 agg_sh, cnt_sh, sem):
    cid = lax.axis_index("c")
    sid = lax.axis_index("s")
    w = cid * NS + sid

    _zero_fill_1d(zrow_v, RPT)
    _zero_fill_2d(rows_bufs[0], B)

    def fill_ones(j, _):
        ones_v[pl.ds(j * 16, 16)] = jnp.ones((16,), jnp.float32)
        return 0
    lax.fori_loop(0, B // 16, fill_ones, 0)

    rbase = sid * RPT
    for k in range(RPT // B):
        pltpu.sync_copy(rows_bufs[0], agg_sh.at[pl.ds(rbase + k * B, B)])
    pltpu.sync_copy(zrow_v, cnt_sh.at[pl.ds(rbase, RPT)])
    plsc.subcore_barrier()

    nch = jnp.where(cid == 0, NBAT0 // CHUNK, NBAT1 // CHUNK)
    rb = jnp.where(cid == 0, sid * NBAT0, NROWS0 + sid * NBAT1)

    def chunk_body(ch, _):
        r0 = rb + ch * CHUNK
        pltpu.sync_copy(src_hbm.at[pl.ds(r0, CHUNK)], src_v)
        pltpu.sync_copy(dst_hbm.at[pl.ds(r0, CHUNK)], dst_v)
        for b in range(RING):
            pltpu.async_copy(x_hbm.at[src_v.at[b]], rows_bufs[b], sem)

        def inner(g, _):
            for b in range(RING):
                j = g * RING + b
                pltpu.make_async_copy(
                    x_hbm.at[src_v.at[j]], rows_bufs[b], sem).wait()
                pltpu.sync_copy(rows_bufs[b], agg_sh.at[dst_v.at[j]],
                                add=True)
                pltpu.sync_copy(ones_v, cnt_sh.at[dst_v.at[j]], add=True)

                @pl.when(g < CHUNK // RING - 1)
                def _():
                    pltpu.async_copy(
                        x_hbm.at[src_v.at[j + RING]], rows_bufs[b], sem)
            return 0
        lax.fori_loop(0, CHUNK // RING, inner, 0)
        return 0
    lax.fori_loop(0, nch, chunk_body, 0)
    plsc.subcore_barrier()

    pltpu.sync_copy(agg_sh.at[pl.ds(rbase, RPT)],
                    agg_out.at[cid, pl.ds(rbase, RPT)])
    pltpu.sync_copy(cnt_sh.at[pl.ds(rbase, RPT)],
                    cnt_out.at[cid, pl.ds(rbase, RPT)])


@functools.partial(
    pl.kernel,
    out_type=jax.ShapeDtypeStruct((NP,), jnp.float32),
    mesh=_mesh,
    scratch_types=(
        pltpu.VMEM((NBAT_S, B), jnp.int32),
        pltpu.VMEM((NBAT_S, B), jnp.int32),
        tuple(pltpu.VMEM((B,), jnp.float32) for _ in range(RING2)),
        pltpu.VMEM((RPT,), jnp.float32),
        pltpu.VMEM((RPT,), jnp.float32),
        pltpu.VMEM((RPT,), jnp.float32),
        pltpu.VMEM((RPT,), jnp.float32),
        pltpu.VMEM_SHARED((NP,), jnp.float32),
        pltpu.SemaphoreType.DMA,
    ),
)
def _sc_agg_scalar(src_hbm, dst_hbm, s_hbm, inv_hbm, r_hbm, out_hbm,
                   src_v, dst_v, val_bufs, acc_v, iv_v, rr_v, out_v,
                   agg_sh, sem):
    cid = lax.axis_index("c")
    sid = lax.axis_index("s")

    @pl.when(cid == 0)
    def _():
        _zero_fill_1d(acc_v, RPT)

        pltpu.sync_copy(src_hbm.at[pl.ds(sid * NBAT_S, NBAT_S)], src_v)
        pltpu.sync_copy(dst_hbm.at[pl.ds(sid * NBAT_S, NBAT_S)], dst_v)

        rbase = sid * RPT
        pltpu.sync_copy(acc_v, agg_sh.at[pl.ds(rbase, RPT)])
        plsc.subcore_barrier()

        for b in range(RING2):
            pltpu.async_copy(s_hbm.at[src_v.at[b]], val_bufs[b], sem)

        def outer(g, _):
            for b in range(RING2):
                j = g * RING2 + b
                pltpu.make_async_copy(
                    s_hbm.at[src_v.at[j]], val_bufs[b], sem).wait()
                pltpu.sync_copy(val_bufs[b], agg_sh.at[dst_v.at[j]],
                                add=True)

                @pl.when(g < NOUT2 - 1)
                def _():
                    pltpu.async_copy(
                        s_hbm.at[src_v.at[j + RING2]], val_bufs[b], sem)
            return 0
        lax.fori_loop(0, NOUT2, outer, 0)
        plsc.subcore_barrier()

        pltpu.sync_copy(agg_sh.at[pl.ds(rbase, RPT)], acc_v)
        pltpu.sync_copy(inv_hbm.at[pl.ds(rbase, RPT)], iv_v)
        pltpu.sync_copy(r_hbm.at[pl.ds(rbase, RPT)], rr_v)

        def combine(k, _):
            sl = pl.ds(k * 16, 16)
            out_v[sl] = acc_v[sl] * iv_v[sl] + rr_v[sl]
            return 0
        lax.fori_loop(0, RPT // 16, combine, 0)
        pltpu.sync_copy(out_v, out_hbm.at[pl.ds(rbase, RPT)])


_RB = 512


def _dense_body(agg0_ref, agg1_ref, x_ref, cnt0_ref, cnt1_ref,
                w1l_ref, w1r_ref, b1_ref, w2_ref, b2_ref,
                sr_ref, inv_ref):
    cnt = cnt0_ref[...] + cnt1_ref[...]
    inv = 1.0 / jnp.maximum(cnt, 1.0)
    mean = (agg0_ref[...] + agg1_ref[...]) * inv
    h = jnp.maximum(
        jnp.dot(mean, w1l_ref[...], preferred_element_type=jnp.float32)
        + jnp.dot(x_ref[...], w1r_ref[...], preferred_element_type=jnp.float32)
        + b1_ref[...],
        0.0,
    )
    sr_ref[...] = (
        jnp.dot(h, w2_ref[...], preferred_element_type=jnp.float32)
        + b2_ref[...]
    )
    inv_ref[...] = inv


def _dense(agg0, agg1, x, cnt0, cnt1, w1l, w1r, b1, w2, b2):
    grid = (NP // _RB,)
    return pl.pallas_call(
        _dense_body,
        grid=grid,
        in_specs=[
            pl.BlockSpec((_RB, D), lambda i: (i, 0)),
            pl.BlockSpec((_RB, D), lambda i: (i, 0)),
            pl.BlockSpec((_RB, D), lambda i: (i, 0)),
            pl.BlockSpec((_RB, 1), lambda i: (i, 0)),
            pl.BlockSpec((_RB, 1), lambda i: (i, 0)),
            pl.BlockSpec((D, D), lambda i: (0, 0)),
            pl.BlockSpec((D, D), lambda i: (0, 0)),
            pl.BlockSpec((1, D), lambda i: (0, 0)),
            pl.BlockSpec((D, 2), lambda i: (0, 0)),
            pl.BlockSpec((1, 2), lambda i: (0, 0)),
        ],
        out_specs=[
            pl.BlockSpec((_RB, 2), lambda i: (i, 0)),
            pl.BlockSpec((_RB, 1), lambda i: (i, 0)),
        ],
        out_shape=[
            jax.ShapeDtypeStruct((NP, 2), jnp.float32),
            jax.ShapeDtypeStruct((NP, 1), jnp.float32),
        ],
    )(agg0, agg1, x, cnt0, cnt1, w1l, w1r, b1, w2, b2)


@jax.jit
def kernel(x, edge_index, W1l, W1r, b1, W2l, W2r, b2):
    xp = jnp.pad(x, ((0, NP - N), (0, 0)))
    pad_dst = N + jnp.arange(EP - E, dtype=jnp.int32) % (NP - N)
    src = jnp.pad(edge_index[0], (0, EP - E)).reshape(EROWS, B)
    dst = jnp.concatenate([edge_index[1], pad_dst]).reshape(EROWS, B)

    aggx, cnt = _sc_agg_wide(src, dst, xp)

    w2 = jnp.concatenate([W2l, W2r], axis=1)
    b2r = jnp.concatenate([jnp.zeros((1,), b2.dtype), b2]).reshape(1, 2)

    sr, inv = _dense(
        aggx[0], aggx[1], xp,
        cnt[0].reshape(NP, 1), cnt[1].reshape(NP, 1),
        W1l, W1r, b1.reshape(1, D), w2, b2r,
    )

    s = sr[:, 0].reshape(NP)
    r = sr[:, 1].reshape(NP)
    out = _sc_agg_scalar(src, dst, s, inv.reshape(NP), r)
    return out[:N]


# speedup vs baseline: 2.5520x; 2.5520x over previous
import functools

import jax
import jax.numpy as jnp
from jax import lax
from jax.experimental import pallas as pl
from jax.experimental.pallas import tpu as pltpu
from jax.experimental.pallas import tpu_sc as plsc

N = 10000
E = 320000
D = 128
NP = 10240
NC = 2
NS = 16
NW = NC * NS
B = 128
EROWS = 2560
EP = EROWS * B
RPT = NP // NS
RING = 2
CHUNK = 16
NBAT0 = 80
NBAT1 = 80
NROWS0 = NS * NBAT0
NBAT_S = 160
RING2 = 8
NOUT2 = NBAT_S // RING2

_mesh = plsc.VectorSubcoreMesh(core_axis_name="c", subcore_axis_name="s")


def _zero_fill_2d(ref, rows):
    def row(i, _):
        def col(j, _):
            ref[i, pl.ds(j * 16, 16)] = jnp.zeros((16,), jnp.float32)
            return 0
        return lax.fori_loop(0, D // 16, col, 0)
    lax.fori_loop(0, rows, row, 0)


def _zero_fill_1d(ref, n):
    def body(j, _):
        ref[pl.ds(j * 16, 16)] = jnp.zeros((16,), jnp.float32)
        return 0
    lax.fori_loop(0, n // 16, body, 0)


@functools.partial(
    pl.kernel,
    out_type=(
        jax.ShapeDtypeStruct((NC, NP, D), jnp.float32),
        jax.ShapeDtypeStruct((NC, NP), jnp.float32),
    ),
    mesh=_mesh,
    scratch_types=(
        pltpu.VMEM((CHUNK, B), jnp.int32),
        pltpu.VMEM((CHUNK, B), jnp.int32),
        tuple(pltpu.VMEM((B, D), jnp.float32) for _ in range(RING)),
        pltpu.VMEM((B,), jnp.float32),
        pltpu.VMEM((RPT,), jnp.float32),
        pltpu.VMEM_SHARED((NP, D), jnp.float32),
        pltpu.VMEM_SHARED((NP,), jnp.float32),
        pltpu.SemaphoreType.DMA,
    ),
)
def _sc_agg_wide(src_hbm, dst_hbm, x_hbm, agg_out, cnt_out,
                 src_v, dst_v, rows_bufs, ones_v, zrow_v,
                 agg_sh, cnt_sh, sem):
    cid = lax.axis_index("c")
    sid = lax.axis_index("s")
    w = cid * NS + sid

    _zero_fill_1d(zrow_v, RPT)
    _zero_fill_2d(rows_bufs[0], B)

    def fill_ones(j, _):
        ones_v[pl.ds(j * 16, 16)] = jnp.ones((16,), jnp.float32)
        return 0
    lax.fori_loop(0, B // 16, fill_ones, 0)

    rbase = sid * RPT
    for k in range(RPT // B):
        pltpu.sync_copy(rows_bufs[0], agg_sh.at[pl.ds(rbase + k * B, B)])
    pltpu.sync_copy(zrow_v, cnt_sh.at[pl.ds(rbase, RPT)])
    plsc.subcore_barrier()

    nch = jnp.where(cid == 0, NBAT0 // CHUNK, NBAT1 // CHUNK)
    rb = jnp.where(cid == 0, sid * NBAT0, NROWS0 + sid * NBAT1)

    def chunk_body(ch, _):
        r0 = rb + ch * CHUNK
        pltpu.sync_copy(src_hbm.at[pl.ds(r0, CHUNK)], src_v)
        pltpu.sync_copy(dst_hbm.at[pl.ds(r0, CHUNK)], dst_v)
        for b in range(RING):
            pltpu.async_copy(x_hbm.at[src_v.at[b]], rows_bufs[b], sem)

        def inner(g, _):
            for b in range(RING):
                j = g * RING + b
                pltpu.make_async_copy(
                    x_hbm.at[src_v.at[j]], rows_bufs[b], sem).wait()
                pltpu.sync_copy(rows_bufs[b], agg_sh.at[dst_v.at[j]],
                                add=True)
                pltpu.sync_copy(ones_v, cnt_sh.at[dst_v.at[j]], add=True)

                @pl.when(g < CHUNK // RING - 1)
                def _():
                    pltpu.async_copy(
                        x_hbm.at[src_v.at[j + RING]], rows_bufs[b], sem)
            return 0
        lax.fori_loop(0, CHUNK // RING, inner, 0)
        return 0
    lax.fori_loop(0, nch, chunk_body, 0)
    plsc.subcore_barrier()

    pltpu.sync_copy(agg_sh.at[pl.ds(rbase, RPT)],
                    agg_out.at[cid, pl.ds(rbase, RPT)])
    pltpu.sync_copy(cnt_sh.at[pl.ds(rbase, RPT)],
                    cnt_out.at[cid, pl.ds(rbase, RPT)])


@functools.partial(
    pl.kernel,
    out_type=jax.ShapeDtypeStruct((NP,), jnp.float32),
    mesh=_mesh,
    scratch_types=(
        pltpu.VMEM((NBAT_S, B), jnp.int32),
        pltpu.VMEM((NBAT_S, B), jnp.int32),
        tuple(pltpu.VMEM((B,), jnp.float32) for _ in range(RING2)),
        pltpu.VMEM((RPT,), jnp.float32),
        pltpu.VMEM((RPT,), jnp.float32),
        pltpu.VMEM((RPT,), jnp.float32),
        pltpu.VMEM((RPT,), jnp.float32),
        pltpu.VMEM_SHARED((NP,), jnp.float32),
        pltpu.SemaphoreType.DMA,
    ),
)
def _sc_agg_scalar(src_hbm, dst_hbm, s_hbm, inv_hbm, r_hbm, out_hbm,
                   src_v, dst_v, val_bufs, acc_v, iv_v, rr_v, out_v,
                   agg_sh, sem):
    cid = lax.axis_index("c")
    sid = lax.axis_index("s")

    @pl.when(cid == 0)
    def _():
        _zero_fill_1d(acc_v, RPT)

        pltpu.sync_copy(src_hbm.at[pl.ds(sid * NBAT_S, NBAT_S)], src_v)
        pltpu.sync_copy(dst_hbm.at[pl.ds(sid * NBAT_S, NBAT_S)], dst_v)

        rbase = sid * RPT
        pltpu.sync_copy(acc_v, agg_sh.at[pl.ds(rbase, RPT)])
        plsc.subcore_barrier()

        for b in range(RING2):
            pltpu.async_copy(s_hbm.at[src_v.at[b]], val_bufs[b], sem)

        def outer(g, _):
            for b in range(RING2):
                j = g * RING2 + b
                pltpu.make_async_copy(
                    s_hbm.at[src_v.at[j]], val_bufs[b], sem).wait()
                pltpu.sync_copy(val_bufs[b], agg_sh.at[dst_v.at[j]],
                                add=True)

                @pl.when(g < NOUT2 - 1)
                def _():
                    pltpu.async_copy(
                        s_hbm.at[src_v.at[j + RING2]], val_bufs[b], sem)
            return 0
        lax.fori_loop(0, NOUT2, outer, 0)
        plsc.subcore_barrier()

        pltpu.sync_copy(agg_sh.at[pl.ds(rbase, RPT)], acc_v)
        pltpu.sync_copy(inv_hbm.at[pl.ds(rbase, RPT)], iv_v)
        pltpu.sync_copy(r_hbm.at[pl.ds(rbase, RPT)], rr_v)

        def combine(k, _):
            sl = pl.ds(k * 16, 16)
            out_v[sl] = acc_v[sl] * iv_v[sl] + rr_v[sl]
            return 0
        lax.fori_loop(0, RPT // 16, combine, 0)
        pltpu.sync_copy(out_v, out_hbm.at[pl.ds(rbase, RPT)])


_RB = 512


def _dense_body(agg0_ref, agg1_ref, x_ref, cnt0_ref, cnt1_ref,
                w1l_ref, w1r_ref, b1_ref, w2_ref, b2_ref,
                sr_ref, inv_ref):
    cnt = cnt0_ref[...] + cnt1_ref[...]
    inv = 1.0 / jnp.maximum(cnt, 1.0)
    mean = (agg0_ref[...] + agg1_ref[...]) * inv
    h = jnp.maximum(
        jnp.dot(mean, w1l_ref[...], preferred_element_type=jnp.float32)
        + jnp.dot(x_ref[...], w1r_ref[...], preferred_element_type=jnp.float32)
        + b1_ref[...],
        0.0,
    )
    sr_ref[...] = (
        jnp.dot(h, w2_ref[...], preferred_element_type=jnp.float32)
        + b2_ref[...]
    )
    inv_ref[...] = inv


def _dense(agg0, agg1, x, cnt0, cnt1, w1l, w1r, b1, w2, b2):
    grid = (NP // _RB,)
    return pl.pallas_call(
        _dense_body,
        grid=grid,
        in_specs=[
            pl.BlockSpec((_RB, D), lambda i: (i, 0)),
            pl.BlockSpec((_RB, D), lambda i: (i, 0)),
            pl.BlockSpec((_RB, D), lambda i: (i, 0)),
            pl.BlockSpec((_RB, 1), lambda i: (i, 0)),
            pl.BlockSpec((_RB, 1), lambda i: (i, 0)),
            pl.BlockSpec((D, D), lambda i: (0, 0)),
            pl.BlockSpec((D, D), lambda i: (0, 0)),
            pl.BlockSpec((1, D), lambda i: (0, 0)),
            pl.BlockSpec((D, 2), lambda i: (0, 0)),
            pl.BlockSpec((1, 2), lambda i: (0, 0)),
        ],
        out_specs=[
            pl.BlockSpec((_RB, 2), lambda i: (i, 0)),
            pl.BlockSpec((_RB, 1), lambda i: (i, 0)),
        ],
        out_shape=[
            jax.ShapeDtypeStruct((NP, 2), jnp.float32),
            jax.ShapeDtypeStruct((NP, 1), jnp.float32),
        ],
    )(agg0, agg1, x, cnt0, cnt1, w1l, w1r, b1, w2, b2)


@jax.jit
def kernel(x, edge_index, W1l, W1r, b1, W2l, W2r, b2):
    xp = jnp.pad(x, ((0, NP - N), (0, 0)))
    npad = EP - E
    pad_src = jnp.arange(npad, dtype=jnp.int32) % N
    pad_dst = N + jnp.arange(npad, dtype=jnp.int32) % (NP - N)
    src = jnp.concatenate([edge_index[0], pad_src]).reshape(EROWS, B)
    dst = jnp.concatenate([edge_index[1], pad_dst]).reshape(EROWS, B)

    aggx, cnt = _sc_agg_wide(src, dst, xp)

    w2 = jnp.concatenate([W2l, W2r], axis=1)
    b2r = jnp.concatenate([jnp.zeros((1,), b2.dtype), b2]).reshape(1, 2)

    sr, inv = _dense(
        aggx[0], aggx[1], xp,
        cnt[0].reshape(NP, 1), cnt[1].reshape(NP, 1),
        W1l, W1r, b1.reshape(1, D), w2, b2r,
    )

    s = sr[:, 0].reshape(NP)
    r = sr[:, 1].reshape(NP)
    out = _sc_agg_scalar(src, dst, s, inv.reshape(NP), r)
    return out[:N]


# speedup vs baseline: 2.6672x; 1.0451x over previous
import functools

import jax
import jax.numpy as jnp
from jax import lax
from jax.experimental import pallas as pl
from jax.experimental.pallas import tpu as pltpu
from jax.experimental.pallas import tpu_sc as plsc

N = 10000
E = 320000
D = 128
NP = 10240
NC = 2
NS = 16
NW = NC * NS
B = 128
EROWS = 2560
EP = EROWS * B
RPT = NP // NS
RING = 2
CHUNK = 16
NBAT0 = 80
NBAT1 = 80
NROWS0 = NS * NBAT0
NBAT_S = 160
RING2 = 16
NOUT2 = NBAT_S // RING2

_mesh = plsc.VectorSubcoreMesh(core_axis_name="c", subcore_axis_name="s")


def _zero_fill_2d(ref, rows):
    def row(i, _):
        def col(j, _):
            ref[i, pl.ds(j * 16, 16)] = jnp.zeros((16,), jnp.float32)
            return 0
        return lax.fori_loop(0, D // 16, col, 0)
    lax.fori_loop(0, rows, row, 0)


def _zero_fill_1d(ref, n):
    def body(j, _):
        ref[pl.ds(j * 16, 16)] = jnp.zeros((16,), jnp.float32)
        return 0
    lax.fori_loop(0, n // 16, body, 0)


@functools.partial(
    pl.kernel,
    out_type=(
        jax.ShapeDtypeStruct((NC, NP, D), jnp.float32),
        jax.ShapeDtypeStruct((NC, NP), jnp.float32),
    ),
    mesh=_mesh,
    scratch_types=(
        pltpu.VMEM((CHUNK, B), jnp.int32),
        pltpu.VMEM((CHUNK, B), jnp.int32),
        tuple(pltpu.VMEM((B, D), jnp.float32) for _ in range(RING)),
        pltpu.VMEM((B,), jnp.float32),
        pltpu.VMEM((RPT,), jnp.float32),
        pltpu.VMEM_SHARED((NP, D), jnp.float32),
        pltpu.VMEM_SHARED((NP,), jnp.float32),
        pltpu.SemaphoreType.DMA,
    ),
)
def _sc_agg_wide(src_hbm, dst_hbm, x_hbm, agg_out, cnt_out,
                 src_v, dst_v, rows_bufs, ones_v, zrow_v,
                 agg_sh, cnt_sh, sem):
    cid = lax.axis_index("c")
    sid = lax.axis_index("s")
    w = cid * NS + sid

    _zero_fill_1d(zrow_v, RPT)
    _zero_fill_2d(rows_bufs[0], B)

    def fill_ones(j, _):
        ones_v[pl.ds(j * 16, 16)] = jnp.ones((16,), jnp.float32)
        return 0
    lax.fori_loop(0, B // 16, fill_ones, 0)

    rbase = sid * RPT
    for k in range(RPT // B):
        pltpu.sync_copy(rows_bufs[0], agg_sh.at[pl.ds(rbase + k * B, B)])
    pltpu.sync_copy(zrow_v, cnt_sh.at[pl.ds(rbase, RPT)])
    plsc.subcore_barrier()

    nch = jnp.where(cid == 0, NBAT0 // CHUNK, NBAT1 // CHUNK)
    rb = jnp.where(cid == 0, sid * NBAT0, NROWS0 + sid * NBAT1)

    def chunk_body(ch, _):
        r0 = rb + ch * CHUNK
        pltpu.sync_copy(src_hbm.at[pl.ds(r0, CHUNK)], src_v)
        pltpu.sync_copy(dst_hbm.at[pl.ds(r0, CHUNK)], dst_v)
        for b in range(RING):
            pltpu.async_copy(x_hbm.at[src_v.at[b]], rows_bufs[b], sem)

        def inner(g, _):
            for b in range(RING):
                j = g * RING + b
                pltpu.make_async_copy(
                    x_hbm.at[src_v.at[j]], rows_bufs[b], sem).wait()
                pltpu.sync_copy(rows_bufs[b], agg_sh.at[dst_v.at[j]],
                                add=True)
                pltpu.sync_copy(ones_v, cnt_sh.at[dst_v.at[j]], add=True)

                @pl.when(g < CHUNK // RING - 1)
                def _():
                    pltpu.async_copy(
                        x_hbm.at[src_v.at[j + RING]], rows_bufs[b], sem)
            return 0
        lax.fori_loop(0, CHUNK // RING, inner, 0)
        return 0
    lax.fori_loop(0, nch, chunk_body, 0)
    plsc.subcore_barrier()

    pltpu.sync_copy(agg_sh.at[pl.ds(rbase, RPT)],
                    agg_out.at[cid, pl.ds(rbase, RPT)])
    pltpu.sync_copy(cnt_sh.at[pl.ds(rbase, RPT)],
                    cnt_out.at[cid, pl.ds(rbase, RPT)])


@functools.partial(
    pl.kernel,
    out_type=jax.ShapeDtypeStruct((NP,), jnp.float32),
    mesh=_mesh,
    scratch_types=(
        pltpu.VMEM((NBAT_S, B), jnp.int32),
        pltpu.VMEM((NBAT_S, B), jnp.int32),
        tuple(pltpu.VMEM((B,), jnp.float32) for _ in range(RING2)),
        pltpu.VMEM((RPT,), jnp.float32),
        pltpu.VMEM((RPT,), jnp.float32),
        pltpu.VMEM((RPT,), jnp.float32),
        pltpu.VMEM((RPT,), jnp.float32),
        pltpu.VMEM_SHARED((NP,), jnp.float32),
        pltpu.SemaphoreType.DMA,
    ),
)
def _sc_agg_scalar(src_hbm, dst_hbm, s_hbm, inv_hbm, r_hbm, out_hbm,
                   src_v, dst_v, val_bufs, acc_v, iv_v, rr_v, out_v,
                   agg_sh, sem):
    cid = lax.axis_index("c")
    sid = lax.axis_index("s")

    @pl.when(cid == 0)
    def _():
        _zero_fill_1d(acc_v, RPT)

        pltpu.sync_copy(src_hbm.at[pl.ds(sid * NBAT_S, NBAT_S)], src_v)
        pltpu.sync_copy(dst_hbm.at[pl.ds(sid * NBAT_S, NBAT_S)], dst_v)

        rbase = sid * RPT
        pltpu.sync_copy(acc_v, agg_sh.at[pl.ds(rbase, RPT)])
        plsc.subcore_barrier()

        for b in range(RING2):
            pltpu.async_copy(s_hbm.at[src_v.at[b]], val_bufs[b], sem)

        def outer(g, _):
            for b in range(RING2):
                j = g * RING2 + b
                pltpu.make_async_copy(
                    s_hbm.at[src_v.at[j]], val_bufs[b], sem).wait()
                pltpu.sync_copy(val_bufs[b], agg_sh.at[dst_v.at[j]],
                                add=True)

                @pl.when(g < NOUT2 - 1)
                def _():
                    pltpu.async_copy(
                        s_hbm.at[src_v.at[j + RING2]], val_bufs[b], sem)
            return 0
        lax.fori_loop(0, NOUT2, outer, 0)
        plsc.subcore_barrier()

        pltpu.sync_copy(agg_sh.at[pl.ds(rbase, RPT)], acc_v)
        pltpu.sync_copy(inv_hbm.at[pl.ds(rbase, RPT)], iv_v)
        pltpu.sync_copy(r_hbm.at[pl.ds(rbase, RPT)], rr_v)

        def combine(k, _):
            sl = pl.ds(k * 16, 16)
            out_v[sl] = acc_v[sl] * iv_v[sl] + rr_v[sl]
            return 0
        lax.fori_loop(0, RPT // 16, combine, 0)
        pltpu.sync_copy(out_v, out_hbm.at[pl.ds(rbase, RPT)])


_RB = 1024


def _dense_body(agg0_ref, agg1_ref, x_ref, cnt0_ref, cnt1_ref,
                w1l_ref, w1r_ref, b1_ref, w2_ref, b2_ref,
                sr_ref, inv_ref):
    cnt = cnt0_ref[...] + cnt1_ref[...]
    inv = 1.0 / jnp.maximum(cnt, 1.0)
    mean = (agg0_ref[...] + agg1_ref[...]) * inv
    h = jnp.maximum(
        jnp.dot(mean, w1l_ref[...], preferred_element_type=jnp.float32)
        + jnp.dot(x_ref[...], w1r_ref[...], preferred_element_type=jnp.float32)
        + b1_ref[...],
        0.0,
    )
    sr_ref[...] = (
        jnp.dot(h, w2_ref[...], preferred_element_type=jnp.float32)
        + b2_ref[...]
    )
    inv_ref[...] = inv


def _dense(agg0, agg1, x, cnt0, cnt1, w1l, w1r, b1, w2, b2):
    grid = (NP // _RB,)
    return pl.pallas_call(
        _dense_body,
        grid=grid,
        in_specs=[
            pl.BlockSpec((_RB, D), lambda i: (i, 0)),
            pl.BlockSpec((_RB, D), lambda i: (i, 0)),
            pl.BlockSpec((_RB, D), lambda i: (i, 0)),
            pl.BlockSpec((_RB, 1), lambda i: (i, 0)),
            pl.BlockSpec((_RB, 1), lambda i: (i, 0)),
            pl.BlockSpec((D, D), lambda i: (0, 0)),
            pl.BlockSpec((D, D), lambda i: (0, 0)),
            pl.BlockSpec((1, D), lambda i: (0, 0)),
            pl.BlockSpec((D, 2), lambda i: (0, 0)),
            pl.BlockSpec((1, 2), lambda i: (0, 0)),
        ],
        out_specs=[
            pl.BlockSpec((_RB, 2), lambda i: (i, 0)),
            pl.BlockSpec((_RB, 1), lambda i: (i, 0)),
        ],
        out_shape=[
            jax.ShapeDtypeStruct((NP, 2), jnp.float32),
            jax.ShapeDtypeStruct((NP, 1), jnp.float32),
        ],
    )(agg0, agg1, x, cnt0, cnt1, w1l, w1r, b1, w2, b2)


@jax.jit
def kernel(x, edge_index, W1l, W1r, b1, W2l, W2r, b2):
    xp = jnp.pad(x, ((0, NP - N), (0, 0)))
    npad = EP - E
    pad_src = jnp.arange(npad, dtype=jnp.int32) % N
    pad_dst = N + jnp.arange(npad, dtype=jnp.int32) % (NP - N)
    src = jnp.concatenate([edge_index[0], pad_src]).reshape(EROWS, B)
    dst = jnp.concatenate([edge_index[1], pad_dst]).reshape(EROWS, B)

    aggx, cnt = _sc_agg_wide(src, dst, xp)

    w2 = jnp.concatenate([W2l, W2r], axis=1)
    b2r = jnp.concatenate([jnp.zeros((1,), b2.dtype), b2]).reshape(1, 2)

    sr, inv = _dense(
        aggx[0], aggx[1], xp,
        cnt[0].reshape(NP, 1), cnt[1].reshape(NP, 1),
        W1l, W1r, b1.reshape(1, D), w2, b2r,
    )

    s = sr[:, 0].reshape(NP)
    r = sr[:, 1].reshape(NP)
    out = _sc_agg_scalar(src, dst, s, inv.reshape(NP), r)
    return out[:N]
